# baseline (speedup 1.0000x reference)
import functools

import jax
import jax.numpy as jnp
from jax import lax
from jax.experimental import pallas as pl
from jax.experimental.pallas import tpu as pltpu
from jax.experimental.pallas import tpu_sc as plsc

BATCH = 16384
EMBED_DIM = 32
GROUP = 4

_info = plsc.get_sparse_core_info()
_NC, _NS = _info.num_cores, _info.num_subcores
_NW = _NC * _NS
_B_PER_W = BATCH // _NW


@functools.partial(
    pl.kernel,
    mesh=plsc.VectorSubcoreMesh(core_axis_name="c", subcore_axis_name="s"),
    out_type=jax.ShapeDtypeStruct((BATCH // GROUP, GROUP * EMBED_DIM), jnp.float32),
    scratch_types=[
        pltpu.VMEM((_B_PER_W,), jnp.int32),
        pltpu.VMEM((_B_PER_W,), jnp.int32),
        pltpu.VMEM((_B_PER_W, GROUP * EMBED_DIM), jnp.float32),
        pltpu.VMEM((_B_PER_W // GROUP, GROUP * EMBED_DIM), jnp.float32),
        pltpu.SemaphoreType.DMA,
    ],
)
def _gather_kernel(x_hbm, table_hbm, out_hbm, idx_v, gidx_v, rows_v, out_v, sem):
    wid = lax.axis_index("s") * _NC + lax.axis_index("c")
    base = wid * _B_PER_W
    pltpu.sync_copy(x_hbm.at[pl.ds(base, _B_PER_W)], idx_v)

    for k in range(_B_PER_W // 16):
        sl = pl.ds(k * 16, 16)
        gidx_v[sl] = lax.shift_right_logical(idx_v[sl], 2)

    pltpu.async_copy(table_hbm.at[gidx_v], rows_v, sem).wait()

    def body(k, _):
        xv = idx_v[pl.ds(k * 16, 16)]
        for l in range(16):
            i = k * 16 + l
            src = (xv[l] & 3) * EMBED_DIM
            di = lax.shift_right_logical(i, 2)
            dc = (i & 3) * EMBED_DIM
            out_v[di, pl.ds(dc, 16)] = rows_v[i, pl.ds(src, 16)]
            out_v[di, pl.ds(dc + 16, 16)] = rows_v[i, pl.ds(src + 16, 16)]
        return 0

    lax.fori_loop(0, _B_PER_W // 16, body, 0)

    pltpu.sync_copy(out_v, out_hbm.at[pl.ds(wid * (_B_PER_W // GROUP), _B_PER_W // GROUP)])


def kernel(x, table):
    table2 = table.reshape(table.shape[0] // GROUP, GROUP * EMBED_DIM)
    out2 = _gather_kernel(x.astype(jnp.int32), table2)
    return out2.reshape(BATCH, EMBED_DIM)
